# baseline (speedup 1.0000x reference)
import functools

import jax
import jax.numpy as jnp
from jax import lax
from jax.experimental import pallas as pl
from jax.experimental.pallas import tpu as pltpu
from jax.experimental.pallas import tpu_sc as plsc

E = 8
D = 1024
H = 4 * D
T = 4096
A2 = 2 * T
TILE = 128
NTILES = A2 // TILE + E
SMAX = NTILES * TILE
NC, NS, L = 2, 16, 16
NW = NC * NS
DW = D // 2


def _routing_body(xf_ref, gw_ref, gb_ref, slot_ref, gate_ref, te_ref,
                  xfb_ref):
    xf = xf_ref[...]
    gw = gw_ref[...]
    logits = lax.dot_general(gw, xf, (((1,), (1,)), ((), ())),
                             preferred_element_type=jnp.float32)
    logits = logits + gb_ref[...]
    eidx = lax.broadcasted_iota(jnp.int32, (E, T), 0)
    v1 = jnp.max(logits, axis=0, keepdims=True)
    i1 = jnp.min(jnp.where(logits == v1, eidx, E), axis=0, keepdims=True)
    masked = jnp.where(eidx == i1, jnp.float32(-1e30), logits)
    v2 = jnp.max(masked, axis=0, keepdims=True)
    i2 = jnp.min(jnp.where(masked == v2, eidx, E), axis=0, keepdims=True)
    e2 = jnp.exp(v2 - v1)
    den = 1.0 + e2
    gate_ref[...] = jnp.concatenate([1.0 / den, e2 / den], axis=1)
    oh = jnp.concatenate([(eidx == i1), (eidx == i2)], axis=1)
    oh = oh.astype(jnp.float32)
    c = oh
    sh = 1
    while sh < A2:
        c = c + jnp.concatenate(
            [jnp.zeros((E, sh), jnp.float32), c[:, :-sh]], axis=1)
        sh *= 2
    rank = c - oh
    counts = c[:, A2 - 1:A2]
    sizes = jnp.ceil(counts * (1.0 / TILE)) * TILE
    cs = sizes
    sh = 1
    while sh < E:
        cs = cs + jnp.concatenate(
            [jnp.zeros((sh, 1), jnp.float32), cs[:-sh, :]], axis=0)
        sh *= 2
    off = cs - sizes
    slot_f = jnp.sum(oh * (off + rank), axis=0, keepdims=True)
    slot_ref[...] = slot_f.astype(jnp.int32)
    starts = (lax.broadcasted_iota(jnp.int32, (1, NTILES), 1)
              * TILE).astype(jnp.float32)
    te_f = jnp.sum((starts >= cs).astype(jnp.float32), axis=0, keepdims=True)
    te_ref[...] = jnp.minimum(te_f, float(E - 1)).astype(jnp.int32)
    xl = lax.bitcast_convert_type(
        xf[:, :DW].astype(jnp.bfloat16), jnp.int16).astype(jnp.int32)
    xh = lax.bitcast_convert_type(
        xf[:, DW:].astype(jnp.bfloat16), jnp.int16).astype(jnp.int32)
    xfb_ref[...] = (xl & 0xFFFF) | (xh << 16)


def _routing(xf, gw, gb):
    return pl.pallas_call(
        _routing_body,
        out_shape=[
            jax.ShapeDtypeStruct((1, A2), jnp.int32),
            jax.ShapeDtypeStruct((1, A2), jnp.float32),
            jax.ShapeDtypeStruct((1, NTILES), jnp.int32),
            jax.ShapeDtypeStruct((T, DW), jnp.int32),
        ],
    )(xf, gw, gb)


_GCH = 32
_GNB = 4
_GPW = SMAX // NW
_ACH = 16
_APW = T // NW


def _wid():
    return lax.axis_index("s") * NC + lax.axis_index("c")


@functools.lru_cache(maxsize=1)
def _sc_kernels():
    mesh = plsc.VectorSubcoreMesh(core_axis_name="c", subcore_axis_name="s",
                                  num_cores=NC, num_subcores=NS)

    @functools.partial(
        pl.kernel,
        out_type=(jax.ShapeDtypeStruct((SMAX,), jnp.int32),
                  jax.ShapeDtypeStruct((SMAX,), jnp.float32)),
        mesh=mesh,
        compiler_params=pltpu.CompilerParams(needs_layout_passes=False),
        scratch_types=[pltpu.VMEM((A2,), jnp.int32),
                       pltpu.VMEM((A2,), jnp.int32),
                       pltpu.VMEM((A2,), jnp.float32),
                       pltpu.VMEM((SMAX,), jnp.int32),
                       pltpu.VMEM((SMAX,), jnp.float32)],
    )
    def build_maps(slot_hbm, tok_hbm, gate_hbm, stok_hbm, sgate_hbm,
                   slot_v, tok_v, gate_v, stok_v, sgate_v):
        wid = _wid()

        @pl.when(wid == 0)
        def _():
            pltpu.sync_copy(slot_hbm, slot_v)
            pltpu.sync_copy(tok_hbm, tok_v)

            def zero(i, carry):
                stok_v[pl.ds(i * L, L)] = jnp.zeros((L,), jnp.int32)
                return carry

            lax.fori_loop(0, SMAX // L, zero, 0)

            def scat(i, carry):
                sl = slot_v[pl.ds(i * L, L)]
                plsc.store_scatter(stok_v, [sl], tok_v[pl.ds(i * L, L)])
                return carry

            lax.fori_loop(0, A2 // L, scat, 0)
            pltpu.sync_copy(stok_v, stok_hbm)

        @pl.when(wid == 1)
        def _():
            pltpu.sync_copy(slot_hbm, slot_v)
            pltpu.sync_copy(gate_hbm, gate_v)

            def zero(i, carry):
                sgate_v[pl.ds(i * L, L)] = jnp.zeros((L,), jnp.float32)
                return carry

            lax.fori_loop(0, SMAX // L, zero, 0)

            def scat(i, carry):
                sl = slot_v[pl.ds(i * L, L)]
                plsc.store_scatter(sgate_v, [sl], gate_v[pl.ds(i * L, L)])
                return carry

            lax.fori_loop(0, A2 // L, scat, 0)
            pltpu.sync_copy(sgate_v, sgate_hbm)

    @functools.partial(
        pl.kernel,
        out_type=jax.ShapeDtypeStruct((SMAX, DW), jnp.int32),
        mesh=mesh,
        compiler_params=pltpu.CompilerParams(needs_layout_passes=False),
        scratch_types=[pltpu.VMEM((_GPW,), jnp.int32)]
                      + [pltpu.VMEM((_GCH, DW), jnp.int32)] * _GNB
                      + [pltpu.SemaphoreType.DMA] * (2 * _GNB),
    )
    def gather_rows(xfb_hbm, stok_hbm, xs_hbm, idx_v, *bufsems):
        w = _wid()
        base = w * _GPW
        bufs = list(bufsems[:_GNB])
        gsems = list(bufsems[_GNB:2 * _GNB])
        wsems = list(bufsems[2 * _GNB:])
        nch = _GPW // _GCH
        pltpu.sync_copy(stok_hbm.at[pl.ds(base, _GPW)], idx_v)
        gd, wd = [None] * nch, [None] * nch

        def fire(c):
            gd[c] = pltpu.async_copy(
                xfb_hbm.at[idx_v.at[pl.ds(c * _GCH, _GCH)]],
                bufs[c % _GNB], gsems[c % _GNB])

        for c in range(min(_GNB, nch)):
            fire(c)
        for c in range(nch):
            gd[c].wait()
            wd[c] = pltpu.async_copy(
                bufs[c % _GNB], xs_hbm.at[pl.ds(base + c * _GCH, _GCH)],
                wsems[c % _GNB])
            if c + _GNB < nch:
                wd[c].wait()
                fire(c + _GNB)
        for c in range(max(0, nch - _GNB), nch):
            wd[c].wait()

    @functools.partial(
        pl.kernel,
        out_type=jax.ShapeDtypeStruct((T, D), jnp.float32),
        mesh=mesh,
        compiler_params=pltpu.CompilerParams(needs_layout_passes=False),
        scratch_types=[pltpu.VMEM((_APW,), jnp.int32),
                       pltpu.VMEM((_APW,), jnp.int32)]
                      + [pltpu.VMEM((_ACH, D), jnp.float32)] * 4
                      + [pltpu.SemaphoreType.DMA] * 6,
    )
    def assemble(ys_hbm, slot_hbm, y_hbm, i0_v, i1_v, r0a, r0b, r1a, r1b,
                 ga, gb, ha, hb, ua, ub):
        w = _wid()
        base = w * _APW
        r0, r1 = [r0a, r0b], [r1a, r1b]
        g0s, g1s, wsems = [ga, gb], [ha, hb], [ua, ub]
        nch = _APW // _ACH
        pltpu.sync_copy(slot_hbm.at[pl.ds(base, _APW)], i0_v)
        pltpu.sync_copy(slot_hbm.at[pl.ds(T + base, _APW)], i1_v)
        gd0, gd1, wd = [None] * nch, [None] * nch, [None] * nch

        def fire(c):
            s = c % 2
            gd0[c] = pltpu.async_copy(
                ys_hbm.at[i0_v.at[pl.ds(c * _ACH, _ACH)]], r0[s], g0s[s])
            gd1[c] = pltpu.async_copy(
                ys_hbm.at[i1_v.at[pl.ds(c * _ACH, _ACH)]], r1[s], g1s[s])

        fire(0)
        for c in range(nch):
            s = c % 2
            gd0[c].wait()
            gd1[c].wait()
            if c + 1 < nch:
                if c >= 1:
                    wd[c - 1].wait()
                fire(c + 1)

            def row(i, rcarry):
                for q in range(D // L):
                    sl = pl.ds(q * L, L)
                    r0[s][i, sl] = r0[s][i, sl] + r1[s][i, sl]
                return rcarry

            lax.fori_loop(0, _ACH, row, 0)
            wd[c] = pltpu.async_copy(
                r0[s], y_hbm.at[pl.ds(base + c * _ACH, _ACH)], wsems[s])
        wd[nch - 2].wait()
        wd[nch - 1].wait()

    return build_maps, gather_rows, assemble


def _ffn_body(te_ref, xs_ref, w1_ref, b1_ref, w2_ref, b2_ref, g_ref, ys_ref,
              w1_vmem, w1_sem):
    j = pl.program_id(0)
    e = te_ref[j]
    prev = jnp.where(j > 0, te_ref[jnp.maximum(j - 1, 0)], -1)

    @pl.when(e != prev)
    def _():
        pltpu.make_async_copy(w1_ref.at[e], w1_vmem, w1_sem).start()
        pltpu.make_async_copy(w1_ref.at[e], w1_vmem, w1_sem).wait()

    xi = xs_ref[...]
    lo = lax.bitcast_convert_type(xi << 16, jnp.float32)
    hi = lax.bitcast_convert_type(xi & jnp.int32(-65536), jnp.float32)
    xb = jnp.concatenate([lo, hi], axis=1)
    h = lax.dot_general(xb, w1_vmem[...], (((1,), (1,)), ((), ())),
                        preferred_element_type=jnp.float32)
    h = h + b1_ref[0]
    h = 0.5 * h * (1.0 + lax.erf(h * 0.7071067811865476))
    o = lax.dot_general(h, w2_ref[0], (((1,), (1,)), ((), ())),
                        preferred_element_type=jnp.float32)
    ys_ref[...] = (o + b2_ref[0]) * g_ref[...]


def _ffn(te, xs, w1b, b1r, w2b, b2r, gcol):
    grid_spec = pltpu.PrefetchScalarGridSpec(
        num_scalar_prefetch=1,
        grid=(NTILES,),
        in_specs=[
            pl.BlockSpec((TILE, DW), lambda j, te: (j, 0)),
            pl.BlockSpec(memory_space=pltpu.MemorySpace.HBM),
            pl.BlockSpec((1, 1, H), lambda j, te: (te[j], 0, 0)),
            pl.BlockSpec((1, D, H), lambda j, te: (te[j], 0, 0)),
            pl.BlockSpec((1, 1, D), lambda j, te: (te[j], 0, 0)),
            pl.BlockSpec((TILE, 1), lambda j, te: (j, 0)),
        ],
        out_specs=pl.BlockSpec((TILE, D), lambda j, te: (j, 0)),
        scratch_shapes=[pltpu.VMEM((H, D), jnp.float32),
                        pltpu.SemaphoreType.DMA],
    )
    return pl.pallas_call(
        _ffn_body,
        grid_spec=grid_spec,
        compiler_params=pltpu.CompilerParams(
            vmem_limit_bytes=112 * 1024 * 1024),
        out_shape=jax.ShapeDtypeStruct((SMAX, D), jnp.float32),
    )(te, xs, w1b, b1r, w2b, b2r, gcol)


def kernel(x, gate_w, gate_b, w1, b1, w2, b2):
    xf = x.reshape(T, D)
    slot2, gate2, te2, xfb = _routing(xf, gate_w, gate_b.reshape(E, 1))
    slot = slot2.reshape(A2)
    gates = gate2.reshape(A2)
    te = te2.reshape(NTILES)
    tok = jnp.arange(A2, dtype=jnp.int32) & (T - 1)
    build_maps, gather_rows, assemble = _sc_kernels()
    stok, sgate = build_maps(slot, tok, gates)
    xs = gather_rows(xfb, stok)
    ys = _ffn(te, xs,
              w1, b1.reshape(E, 1, H),
              w2, b2.reshape(E, 1, D),
              sgate.reshape(SMAX, 1))
    y = assemble(ys, slot)
    return y.reshape(x.shape)


# speedup vs baseline: 1.4050x; 1.4050x over previous
import functools

import jax
import jax.numpy as jnp
from jax import lax
from jax.experimental import pallas as pl
from jax.experimental.pallas import tpu as pltpu
from jax.experimental.pallas import tpu_sc as plsc

E = 8
D = 1024
H = 4 * D
T = 4096
A2 = 2 * T
TILE = 256
NTILES = A2 // TILE + E
SMAX = NTILES * TILE
NC, NS, L = 2, 16, 16
NW = NC * NS
DW = D // 2


def _routing_body(xf_ref, gw_ref, gb_ref, slot_ref, gate_ref, te_ref,
                  xfb_ref):
    xf = xf_ref[...]
    gw = gw_ref[...]
    logits = lax.dot_general(gw, xf, (((1,), (1,)), ((), ())),
                             preferred_element_type=jnp.float32)
    logits = logits + gb_ref[...]
    eidx = lax.broadcasted_iota(jnp.int32, (E, T), 0)
    v1 = jnp.max(logits, axis=0, keepdims=True)
    i1 = jnp.min(jnp.where(logits == v1, eidx, E), axis=0, keepdims=True)
    masked = jnp.where(eidx == i1, jnp.float32(-1e30), logits)
    v2 = jnp.max(masked, axis=0, keepdims=True)
    i2 = jnp.min(jnp.where(masked == v2, eidx, E), axis=0, keepdims=True)
    e2 = jnp.exp(v2 - v1)
    den = 1.0 + e2
    gate_ref[...] = jnp.concatenate([1.0 / den, e2 / den], axis=1)
    oh = jnp.concatenate([(eidx == i1), (eidx == i2)], axis=1)
    oh = oh.astype(jnp.float32)
    c = oh
    sh = 1
    while sh < A2:
        c = c + jnp.concatenate(
            [jnp.zeros((E, sh), jnp.float32), c[:, :-sh]], axis=1)
        sh *= 2
    rank = c - oh
    counts = c[:, A2 - 1:A2]
    sizes = jnp.ceil(counts * (1.0 / TILE)) * TILE
    cs = sizes
    sh = 1
    while sh < E:
        cs = cs + jnp.concatenate(
            [jnp.zeros((sh, 1), jnp.float32), cs[:-sh, :]], axis=0)
        sh *= 2
    off = cs - sizes
    slot_f = jnp.sum(oh * (off + rank), axis=0, keepdims=True)
    slot_ref[...] = slot_f.astype(jnp.int32)
    starts = (lax.broadcasted_iota(jnp.int32, (1, NTILES), 1)
              * TILE).astype(jnp.float32)
    te_f = jnp.sum((starts >= cs).astype(jnp.float32), axis=0, keepdims=True)
    te_ref[...] = jnp.minimum(te_f, float(E - 1)).astype(jnp.int32)
    xl = lax.bitcast_convert_type(
        xf[:, :DW].astype(jnp.bfloat16), jnp.int16).astype(jnp.int32)
    xh = lax.bitcast_convert_type(
        xf[:, DW:].astype(jnp.bfloat16), jnp.int16).astype(jnp.int32)
    xfb_ref[...] = (xl & 0xFFFF) | (xh << 16)


def _routing(xf, gw, gb):
    return pl.pallas_call(
        _routing_body,
        out_shape=[
            jax.ShapeDtypeStruct((1, A2), jnp.int32),
            jax.ShapeDtypeStruct((1, A2), jnp.float32),
            jax.ShapeDtypeStruct((1, NTILES), jnp.int32),
            jax.ShapeDtypeStruct((T, DW), jnp.int32),
        ],
    )(xf, gw, gb)


_GCH = 32
_GNB = 4
_GPW = SMAX // NW
_ACH = 16
_APW = T // NW


def _wid():
    return lax.axis_index("s") * NC + lax.axis_index("c")


@functools.lru_cache(maxsize=1)
def _sc_kernels():
    mesh = plsc.VectorSubcoreMesh(core_axis_name="c", subcore_axis_name="s",
                                  num_cores=NC, num_subcores=NS)

    @functools.partial(
        pl.kernel,
        out_type=(jax.ShapeDtypeStruct((SMAX,), jnp.int32),
                  jax.ShapeDtypeStruct((SMAX,), jnp.float32)),
        mesh=mesh,
        compiler_params=pltpu.CompilerParams(needs_layout_passes=False),
        scratch_types=[pltpu.VMEM((A2,), jnp.int32),
                       pltpu.VMEM((A2,), jnp.int32),
                       pltpu.VMEM((A2,), jnp.float32),
                       pltpu.VMEM((SMAX,), jnp.int32),
                       pltpu.VMEM((SMAX,), jnp.float32)],
    )
    def build_maps(slot_hbm, tok_hbm, gate_hbm, stok_hbm, sgate_hbm,
                   slot_v, tok_v, gate_v, stok_v, sgate_v):
        wid = _wid()

        @pl.when(wid == 0)
        def _():
            pltpu.sync_copy(slot_hbm, slot_v)
            pltpu.sync_copy(tok_hbm, tok_v)

            def zero(i, carry):
                stok_v[pl.ds(i * L, L)] = jnp.zeros((L,), jnp.int32)
                return carry

            lax.fori_loop(0, SMAX // L, zero, 0)

            def scat(i, carry):
                sl = slot_v[pl.ds(i * L, L)]
                plsc.store_scatter(stok_v, [sl], tok_v[pl.ds(i * L, L)])
                return carry

            lax.fori_loop(0, A2 // L, scat, 0)
            pltpu.sync_copy(stok_v, stok_hbm)

        @pl.when(wid == 1)
        def _():
            pltpu.sync_copy(slot_hbm, slot_v)
            pltpu.sync_copy(gate_hbm, gate_v)

            def zero(i, carry):
                sgate_v[pl.ds(i * L, L)] = jnp.zeros((L,), jnp.float32)
                return carry

            lax.fori_loop(0, SMAX // L, zero, 0)

            def scat(i, carry):
                sl = slot_v[pl.ds(i * L, L)]
                plsc.store_scatter(sgate_v, [sl], gate_v[pl.ds(i * L, L)])
                return carry

            lax.fori_loop(0, A2 // L, scat, 0)
            pltpu.sync_copy(sgate_v, sgate_hbm)

    @functools.partial(
        pl.kernel,
        out_type=jax.ShapeDtypeStruct((SMAX, DW), jnp.int32),
        mesh=mesh,
        compiler_params=pltpu.CompilerParams(needs_layout_passes=False),
        scratch_types=[pltpu.VMEM((_GPW,), jnp.int32)]
                      + [pltpu.VMEM((_GCH, DW), jnp.int32)] * _GNB
                      + [pltpu.SemaphoreType.DMA] * (2 * _GNB),
    )
    def gather_rows(xfb_hbm, stok_hbm, xs_hbm, idx_v, *bufsems):
        w = _wid()
        base = w * _GPW
        bufs = list(bufsems[:_GNB])
        gsems = list(bufsems[_GNB:2 * _GNB])
        wsems = list(bufsems[2 * _GNB:])
        nch = _GPW // _GCH
        pltpu.sync_copy(stok_hbm.at[pl.ds(base, _GPW)], idx_v)
        gd, wd = [None] * nch, [None] * nch

        def fire(c):
            gd[c] = pltpu.async_copy(
                xfb_hbm.at[idx_v.at[pl.ds(c * _GCH, _GCH)]],
                bufs[c % _GNB], gsems[c % _GNB])

        for c in range(min(_GNB, nch)):
            fire(c)
        for c in range(nch):
            gd[c].wait()
            wd[c] = pltpu.async_copy(
                bufs[c % _GNB], xs_hbm.at[pl.ds(base + c * _GCH, _GCH)],
                wsems[c % _GNB])
            if c + _GNB < nch:
                wd[c].wait()
                fire(c + _GNB)
        for c in range(max(0, nch - _GNB), nch):
            wd[c].wait()

    @functools.partial(
        pl.kernel,
        out_type=jax.ShapeDtypeStruct((T, D), jnp.float32),
        mesh=mesh,
        compiler_params=pltpu.CompilerParams(needs_layout_passes=False),
        scratch_types=[pltpu.VMEM((_APW,), jnp.int32),
                       pltpu.VMEM((_APW,), jnp.int32)]
                      + [pltpu.VMEM((_ACH, D), jnp.float32)] * 4
                      + [pltpu.SemaphoreType.DMA] * 6,
    )
    def assemble(ys_hbm, slot_hbm, y_hbm, i0_v, i1_v, r0a, r0b, r1a, r1b,
                 ga, gb, ha, hb, ua, ub):
        w = _wid()
        base = w * _APW
        r0, r1 = [r0a, r0b], [r1a, r1b]
        g0s, g1s, wsems = [ga, gb], [ha, hb], [ua, ub]
        nch = _APW // _ACH
        pltpu.sync_copy(slot_hbm.at[pl.ds(base, _APW)], i0_v)
        pltpu.sync_copy(slot_hbm.at[pl.ds(T + base, _APW)], i1_v)
        gd0, gd1, wd = [None] * nch, [None] * nch, [None] * nch

        def fire(c):
            s = c % 2
            gd0[c] = pltpu.async_copy(
                ys_hbm.at[i0_v.at[pl.ds(c * _ACH, _ACH)]], r0[s], g0s[s])
            gd1[c] = pltpu.async_copy(
                ys_hbm.at[i1_v.at[pl.ds(c * _ACH, _ACH)]], r1[s], g1s[s])

        fire(0)
        for c in range(nch):
            s = c % 2
            gd0[c].wait()
            gd1[c].wait()
            if c + 1 < nch:
                if c >= 1:
                    wd[c - 1].wait()
                fire(c + 1)

            def row(i, rcarry):
                for q in range(D // L):
                    sl = pl.ds(q * L, L)
                    r0[s][i, sl] = r0[s][i, sl] + r1[s][i, sl]
                return rcarry

            lax.fori_loop(0, _ACH, row, 0)
            wd[c] = pltpu.async_copy(
                r0[s], y_hbm.at[pl.ds(base + c * _ACH, _ACH)], wsems[s])
        wd[nch - 2].wait()
        wd[nch - 1].wait()

    return build_maps, gather_rows, assemble


def _ffn_body(te_ref, xs_ref, w1_ref, b1_ref, w2_ref, b2_ref, g_ref, ys_ref,
              w1_vmem, w1_sem):
    j = pl.program_id(0)
    e = te_ref[j]
    prev = jnp.where(j > 0, te_ref[jnp.maximum(j - 1, 0)], -1)

    @pl.when(e != prev)
    def _():
        pltpu.make_async_copy(w1_ref.at[e], w1_vmem, w1_sem).start()
        pltpu.make_async_copy(w1_ref.at[e], w1_vmem, w1_sem).wait()

    xi = xs_ref[...]
    lo = lax.bitcast_convert_type(xi << 16, jnp.float32)
    hi = lax.bitcast_convert_type(xi & jnp.int32(-65536), jnp.float32)
    xb = jnp.concatenate([lo, hi], axis=1)
    h = lax.dot_general(xb, w1_vmem[...], (((1,), (1,)), ((), ())),
                        preferred_element_type=jnp.float32)
    h = h + b1_ref[0]
    h = 0.5 * h * (1.0 + lax.erf(h * 0.7071067811865476))
    o = lax.dot_general(h, w2_ref[0], (((1,), (1,)), ((), ())),
                        preferred_element_type=jnp.float32)
    ys_ref[...] = (o + b2_ref[0]) * g_ref[...]


def _ffn(te, xs, w1b, b1r, w2b, b2r, gcol):
    grid_spec = pltpu.PrefetchScalarGridSpec(
        num_scalar_prefetch=1,
        grid=(NTILES,),
        in_specs=[
            pl.BlockSpec((TILE, DW), lambda j, te: (j, 0)),
            pl.BlockSpec(memory_space=pltpu.MemorySpace.HBM),
            pl.BlockSpec((1, 1, H), lambda j, te: (te[j], 0, 0)),
            pl.BlockSpec((1, D, H), lambda j, te: (te[j], 0, 0)),
            pl.BlockSpec((1, 1, D), lambda j, te: (te[j], 0, 0)),
            pl.BlockSpec((TILE, 1), lambda j, te: (j, 0)),
        ],
        out_specs=pl.BlockSpec((TILE, D), lambda j, te: (j, 0)),
        scratch_shapes=[pltpu.VMEM((H, D), jnp.float32),
                        pltpu.SemaphoreType.DMA],
    )
    return pl.pallas_call(
        _ffn_body,
        grid_spec=grid_spec,
        compiler_params=pltpu.CompilerParams(
            vmem_limit_bytes=112 * 1024 * 1024),
        out_shape=jax.ShapeDtypeStruct((SMAX, D), jnp.float32),
    )(te, xs, w1b, b1r, w2b, b2r, gcol)


def kernel(x, gate_w, gate_b, w1, b1, w2, b2):
    xf = x.reshape(T, D)
    slot2, gate2, te2, xfb = _routing(xf, gate_w, gate_b.reshape(E, 1))
    slot = slot2.reshape(A2)
    gates = gate2.reshape(A2)
    te = te2.reshape(NTILES)
    tok = jnp.arange(A2, dtype=jnp.int32) & (T - 1)
    build_maps, gather_rows, assemble = _sc_kernels()
    stok, sgate = build_maps(slot, tok, gates)
    xs = gather_rows(xfb, stok)
    ys = _ffn(te, xs,
              w1, b1.reshape(E, 1, H),
              w2, b2.reshape(E, 1, D),
              sgate.reshape(SMAX, 1))
    y = assemble(ys, slot)
    return y.reshape(x.shape)
